# Initial kernel scaffold; baseline (speedup 1.0000x reference)
import jax
import jax.numpy as jnp
from jax.experimental import pallas as pl


def kernel(h0, coord0, g0, edge_index, batch, gin0_W1, gin0_b1, gin0_gamma, gin0_beta, gin0_W2, gin0_b2, gin1_W1, gin1_b1, gin1_gamma, gin1_beta, gin1_W2, gin1_b2, gin2_W1, gin2_b1, gin2_gamma, gin2_beta, gin2_W2, gin2_b2, cls_W1, cls_b1, cls_gamma, cls_beta, cls_W2, cls_b2):
    raise NotImplementedError("write your pallas kernel here")


# speedup vs baseline: 1.3025x; 1.3025x over previous
import functools

import jax
import jax.numpy as jnp
from jax import lax
from jax.experimental import pallas as pl
from jax.experimental.pallas import tpu as pltpu
from jax.experimental.pallas import tpu_sc as plsc

_N = 10000
_NPAD = 10240
_E = 320000
_HID = 128
_D0 = 144
_NG = 16
_NCLS = 8
_NC = 2
_NS = 16
_NW = _NC * _NS
_RPO = _NPAD // _NW
_CH = 128
_CPT = 88
_G = _NW * _CPT

_F32 = jnp.float32


def _dot(a, b):
    return lax.dot_general(a, b, (((1,), (0,)), ((), ())),
                           preferred_element_type=_F32,
                           precision=lax.Precision.DEFAULT)


def _elu(x):
    return jnp.where(x > 0, x, 0.1 * (jnp.exp(jnp.minimum(x, 0.0)) - 1.0))


def _bf(a):
    while a.shape[0] > 1:
        k = a.shape[0] // 2
        a = a[:k] + a[k:]
    return a


def _sc_segsum(x, e4, zrows):
    d = x.shape[1]
    mesh = plsc.VectorSubcoreMesh(core_axis_name="c", subcore_axis_name="s")

    @functools.partial(
        pl.kernel,
        out_type=jax.ShapeDtypeStruct((_NPAD, d), _F32),
        mesh=mesh,
        compiler_params=pltpu.CompilerParams(
            use_tc_tiling_on_sc=(d % 128 == 0)),
        scratch_types=[
            pltpu.VMEM((2, _CH), jnp.int32),
            pltpu.VMEM((_CH, d), _F32),
            pltpu.VMEM_SHARED((_NPAD // 2, d), _F32),
            pltpu.SemaphoreType.DMA,
        ],
    )
    def k(x_hbm, e_hbm, z_hbm, out_hbm, idx_v, rows_v, acc_sh, sem):
        c = lax.axis_index("c")
        s = lax.axis_index("s")
        w = s * 2 + c
        row0 = s * _RPO
        pltpu.sync_copy(z_hbm, acc_sh.at[pl.ds(row0, _RPO)])
        plsc.subcore_barrier()

        def body(j, carry):
            pltpu.sync_copy(e_hbm.at[w, j], idx_v)
            pltpu.async_copy(x_hbm.at[idx_v.at[0]], rows_v, sem).wait()
            pltpu.sync_copy(rows_v, acc_sh.at[idx_v.at[1]], add=True)
            return carry

        lax.fori_loop(0, _CPT, body, 0)
        plsc.subcore_barrier()
        pltpu.sync_copy(acc_sh.at[pl.ds(row0, _RPO)],
                        out_hbm.at[pl.ds(w * _RPO, _RPO)])

    return k(x, e4, zrows)


def _stats(h_ref):
    def half(lo):
        def step(i, acc):
            return acc + h_ref[pl.ds(lo + i * 8, 8), :]
        return lax.fori_loop(0, 625, step, jnp.zeros((8, _HID), _F32))

    mu = (_bf(half(0)) + _bf(half(5000))) * (1.0 / _N)

    def halfv(lo):
        def step(i, acc):
            dv = h_ref[pl.ds(lo + i * 8, 8), :] - mu
            return acc + dv * dv
        return lax.fori_loop(0, 625, step, jnp.zeros((8, _HID), _F32))

    var = (_bf(halfv(0)) + _bf(halfv(5000))) * (1.0 / _N)
    return mu, var


def _tc_layer(x, agg, w1, b1, gamma, beta, w2, b2):
    def body(x_ref, a_ref, w1_ref, b1_ref, g_ref, be_ref, w2_ref, b2_ref,
             o_ref, h_s):
        hin = x_ref[...] + a_ref[...]
        h_s[...] = _dot(hin, w1_ref[...]) + b1_ref[...]
        mu, var = _stats(h_s)
        hn = (h_s[...] - mu) / jnp.sqrt(var + 1e-5) * g_ref[...] + be_ref[...]
        hn = jnp.maximum(hn, 0.0)
        h2 = _dot(hn, w2_ref[...]) + b2_ref[...]
        rows = lax.broadcasted_iota(jnp.int32, (_NPAD, 1), 0)
        o_ref[...] = jnp.where(rows < _N, h2, 0.0)

    return pl.pallas_call(
        body,
        out_shape=jax.ShapeDtypeStruct((_NPAD, _HID), _F32),
        scratch_shapes=[pltpu.VMEM((_NPAD, _HID), _F32)],
    )(x, agg, w1, b1, gamma, beta, w2, b2)


def _tc_final(x, batch_pad, g0, cw1a, cw1b, cw1c, cb1, cgamma, cbeta,
              cw2, cb2):
    def body(x_ref, batch_ref, g0_ref, w1a_ref, w1b_ref, w1c_ref, cb1_ref,
             cg_ref, cbe_ref, cw2_ref, cb2_ref, o_ref):
        xf = x_ref[...]
        b = batch_ref[...]
        gids = lax.broadcasted_iota(jnp.int32, (_NPAD, _NG), 1)
        oh = (b == gids).astype(_F32)
        cnt = jnp.sum(oh, axis=0, keepdims=True)
        sums = lax.dot_general(oh, xf, (((0,), (0,)), ((), ())),
                               preferred_element_type=_F32,
                               precision=lax.Precision.HIGHEST)
        mp = sums / jnp.maximum(cnt, 1.0).reshape(_NG, 1)
        mx_list = []
        for g in range(_NG):
            sel = jnp.where(b == g, xf, -jnp.inf)
            mx_list.append(jnp.max(sel, axis=0, keepdims=True))
        mx = jnp.concatenate(mx_list, axis=0)

        z = (_dot(mp, w1a_ref[...]) + _dot(mx, w1b_ref[...])
             + _dot(g0_ref[...], w1c_ref[...]) + cb1_ref[...])
        z = _elu(z)
        zmu = _bf(z) * (1.0 / _NG)
        zd = z - zmu
        zvar = _bf(zd * zd) * (1.0 / _NG)
        zn = zd / jnp.sqrt(zvar + 1e-5) * cg_ref[...] + cbe_ref[...]
        logits = _dot(zn, cw2_ref[...]) + cb2_ref[...]
        lmax = jnp.max(logits, axis=1, keepdims=True)
        e = jnp.exp(logits - lmax)
        o_ref[...] = e / jnp.sum(e, axis=1, keepdims=True)

    return pl.pallas_call(
        body,
        out_shape=jax.ShapeDtypeStruct((_NG, _NCLS), _F32),
    )(x, batch_pad, g0, cw1a, cw1b, cw1c, cb1, cgamma, cbeta, cw2, cb2)


def kernel(h0, coord0, g0, edge_index, batch,
           gin0_W1, gin0_b1, gin0_gamma, gin0_beta, gin0_W2, gin0_b2,
           gin1_W1, gin1_b1, gin1_gamma, gin1_beta, gin1_W2, gin1_b2,
           gin2_W1, gin2_b1, gin2_gamma, gin2_beta, gin2_W2, gin2_b2,
           cls_W1, cls_b1, cls_gamma, cls_beta, cls_W2, cls_b2):
    x0 = jnp.concatenate([h0, coord0], axis=1)
    d0 = x0.shape[1]
    x0p = jnp.zeros((_NPAD, _D0), _F32).at[:_N, :d0].set(x0)
    w10 = jnp.zeros((_D0, _HID), _F32).at[:d0].set(gin0_W1)

    src = edge_index[0]
    dst = edge_index[1]
    owner = dst // _RPO
    perm = jnp.argsort(owner, stable=True)
    owner_s = owner[perm]
    src_s = src[perm]
    dst_s = dst[perm]
    dst_local = (owner_s // 2) * _RPO + (dst_s - owner_s * _RPO)

    cnt_w = jnp.bincount(owner, length=_NW)
    est_w = jnp.concatenate([jnp.zeros((1,), cnt_w.dtype),
                             jnp.cumsum(cnt_w)[:-1]])
    rank = jnp.arange(_E) - est_w[owner_s]
    pos = owner_s * (_CPT * _CH) + rank

    slot_owner = jnp.repeat(jnp.arange(_NW), _CPT)
    pad_dst = jnp.repeat((slot_owner // 2) * _RPO, _CH)
    pad_src = (_N + jnp.arange(_G * _CH) % (_NPAD - _N)).astype(jnp.int32)
    flat_src = pad_src.at[pos].set(src_s)
    flat_dst = pad_dst.astype(jnp.int32).at[pos].set(dst_local)
    e4 = jnp.stack([flat_src.reshape(_NW, _CPT, _CH),
                    flat_dst.reshape(_NW, _CPT, _CH)], axis=2)

    zrows0 = jnp.zeros((_RPO, _D0), _F32)
    zrows = jnp.zeros((_RPO, _HID), _F32)
    batch_pad = jnp.full((_NPAD, 1), _NG, jnp.int32).at[:_N, 0].set(batch)

    def row(v):
        return v.reshape(1, -1)

    gins = [
        (w10, gin0_b1, gin0_gamma, gin0_beta, gin0_W2, gin0_b2),
        (gin1_W1, gin1_b1, gin1_gamma, gin1_beta, gin1_W2, gin1_b2),
        (gin2_W1, gin2_b1, gin2_gamma, gin2_beta, gin2_W2, gin2_b2),
    ]

    x = x0p
    for i in range(3):
        w1, b1, gamma, beta, w2, b2 = gins[i]
        agg = _sc_segsum(x, e4, zrows0 if i == 0 else zrows)
        h2 = _tc_layer(x, agg, w1, row(b1), row(gamma), row(beta), w2,
                       row(b2))
        x = jax.nn.elu(h2, alpha=0.1)

    out = _tc_final(
        x, batch_pad, g0,
        cls_W1[:_HID], cls_W1[_HID:2 * _HID], cls_W1[2 * _HID:],
        row(cls_b1), row(cls_gamma), row(cls_beta), cls_W2, row(cls_b2))
    return out


# speedup vs baseline: 1.4035x; 1.0775x over previous
import functools

import jax
import jax.numpy as jnp
from jax import lax
from jax.experimental import pallas as pl
from jax.experimental.pallas import tpu as pltpu
from jax.experimental.pallas import tpu_sc as plsc

_N = 10000
_NPAD = 10240
_E = 320000
_HID = 128
_D0 = 144
_NG = 16
_NCLS = 8
_NC = 2
_NS = 16
_NW = _NC * _NS
_RPO = _NPAD // _NW
_CH = 128
_CPT = 88
_G = _NW * _CPT

_F32 = jnp.float32


def _dot(a, b):
    return lax.dot_general(a, b, (((1,), (0,)), ((), ())),
                           preferred_element_type=_F32,
                           precision=lax.Precision.DEFAULT)


def _elu(x):
    return jnp.where(x > 0, x, 0.1 * (jnp.exp(jnp.minimum(x, 0.0)) - 1.0))


def _bf(a):
    while a.shape[0] > 1:
        k = a.shape[0] // 2
        a = a[:k] + a[k:]
    return a


def _sc_segsum(x, e4, zrows):
    d = x.shape[1]
    mesh = plsc.VectorSubcoreMesh(core_axis_name="c", subcore_axis_name="s")

    @functools.partial(
        pl.kernel,
        out_type=jax.ShapeDtypeStruct((_NPAD, d), _F32),
        mesh=mesh,
        compiler_params=pltpu.CompilerParams(
            use_tc_tiling_on_sc=(d % 128 == 0)),
        scratch_types=[
            pltpu.VMEM((2, 2, _CH), jnp.int32),
            pltpu.VMEM((2, _CH, d), _F32),
            pltpu.VMEM_SHARED((_NPAD // 2, d), _F32),
            pltpu.SemaphoreType.DMA,
        ],
    )
    def k(x_hbm, e_hbm, z_hbm, out_hbm, idx_v, rows_v, acc_sh, sem):
        c = lax.axis_index("c")
        s = lax.axis_index("s")
        w = s * 2 + c
        row0 = s * _RPO
        pltpu.sync_copy(z_hbm, acc_sh.at[pl.ds(row0, _RPO)])
        plsc.subcore_barrier()

        pltpu.sync_copy(e_hbm.at[w, 0], idx_v.at[0])
        pltpu.async_copy(x_hbm.at[idx_v.at[0, 0]], rows_v.at[0], sem)

        def body(j, carry):
            b = lax.rem(j, 2)
            nb = lax.rem(j + 1, 2)

            @pl.when(j + 1 < _CPT)
            def _():
                pltpu.sync_copy(e_hbm.at[w, j + 1], idx_v.at[nb])
                pltpu.async_copy(x_hbm.at[idx_v.at[nb, 0]], rows_v.at[nb],
                                 sem)

            pltpu.make_async_copy(x_hbm.at[idx_v.at[b, 0]], rows_v.at[b],
                                  sem).wait()
            pltpu.sync_copy(rows_v.at[b], acc_sh.at[idx_v.at[b, 1]],
                            add=True)
            return carry

        lax.fori_loop(0, _CPT, body, 0)
        plsc.subcore_barrier()
        pltpu.sync_copy(acc_sh.at[pl.ds(row0, _RPO)],
                        out_hbm.at[pl.ds(w * _RPO, _RPO)])

    return k(x, e4, zrows)


def _stats(h_ref):
    def half(lo):
        def step(i, acc):
            return acc + h_ref[pl.ds(lo + i * 8, 8), :]
        return lax.fori_loop(0, 625, step, jnp.zeros((8, _HID), _F32))

    mu = (_bf(half(0)) + _bf(half(5000))) * (1.0 / _N)

    def halfv(lo):
        def step(i, acc):
            dv = h_ref[pl.ds(lo + i * 8, 8), :] - mu
            return acc + dv * dv
        return lax.fori_loop(0, 625, step, jnp.zeros((8, _HID), _F32))

    var = (_bf(halfv(0)) + _bf(halfv(5000))) * (1.0 / _N)
    return mu, var


def _tc_layer(x, agg, w1, b1, gamma, beta, w2, b2):
    def body(x_ref, a_ref, w1_ref, b1_ref, g_ref, be_ref, w2_ref, b2_ref,
             o_ref, h_s):
        hin = x_ref[...] + a_ref[...]
        h_s[...] = _dot(hin, w1_ref[...]) + b1_ref[...]
        mu, var = _stats(h_s)
        hn = (h_s[...] - mu) / jnp.sqrt(var + 1e-5) * g_ref[...] + be_ref[...]
        hn = jnp.maximum(hn, 0.0)
        h2 = _dot(hn, w2_ref[...]) + b2_ref[...]
        rows = lax.broadcasted_iota(jnp.int32, (_NPAD, 1), 0)
        o_ref[...] = jnp.where(rows < _N, h2, 0.0)

    return pl.pallas_call(
        body,
        out_shape=jax.ShapeDtypeStruct((_NPAD, _HID), _F32),
        scratch_shapes=[pltpu.VMEM((_NPAD, _HID), _F32)],
    )(x, agg, w1, b1, gamma, beta, w2, b2)


def _tc_final(x, batch_pad, g0, cw1a, cw1b, cw1c, cb1, cgamma, cbeta,
              cw2, cb2):
    def body(x_ref, batch_ref, g0_ref, w1a_ref, w1b_ref, w1c_ref, cb1_ref,
             cg_ref, cbe_ref, cw2_ref, cb2_ref, o_ref):
        xf = x_ref[...]
        b = batch_ref[...]
        gids = lax.broadcasted_iota(jnp.int32, (_NPAD, _NG), 1)
        oh = (b == gids).astype(_F32)
        cnt = jnp.sum(oh, axis=0, keepdims=True)
        sums = lax.dot_general(oh, xf, (((0,), (0,)), ((), ())),
                               preferred_element_type=_F32,
                               precision=lax.Precision.HIGHEST)
        mp = sums / jnp.maximum(cnt, 1.0).reshape(_NG, 1)
        mx_list = []
        for g in range(_NG):
            sel = jnp.where(b == g, xf, -jnp.inf)
            mx_list.append(jnp.max(sel, axis=0, keepdims=True))
        mx = jnp.concatenate(mx_list, axis=0)

        z = (_dot(mp, w1a_ref[...]) + _dot(mx, w1b_ref[...])
             + _dot(g0_ref[...], w1c_ref[...]) + cb1_ref[...])
        z = _elu(z)
        zmu = _bf(z) * (1.0 / _NG)
        zd = z - zmu
        zvar = _bf(zd * zd) * (1.0 / _NG)
        zn = zd / jnp.sqrt(zvar + 1e-5) * cg_ref[...] + cbe_ref[...]
        logits = _dot(zn, cw2_ref[...]) + cb2_ref[...]
        lmax = jnp.max(logits, axis=1, keepdims=True)
        e = jnp.exp(logits - lmax)
        o_ref[...] = e / jnp.sum(e, axis=1, keepdims=True)

    return pl.pallas_call(
        body,
        out_shape=jax.ShapeDtypeStruct((_NG, _NCLS), _F32),
    )(x, batch_pad, g0, cw1a, cw1b, cw1c, cb1, cgamma, cbeta, cw2, cb2)


def kernel(h0, coord0, g0, edge_index, batch,
           gin0_W1, gin0_b1, gin0_gamma, gin0_beta, gin0_W2, gin0_b2,
           gin1_W1, gin1_b1, gin1_gamma, gin1_beta, gin1_W2, gin1_b2,
           gin2_W1, gin2_b1, gin2_gamma, gin2_beta, gin2_W2, gin2_b2,
           cls_W1, cls_b1, cls_gamma, cls_beta, cls_W2, cls_b2):
    x0 = jnp.concatenate([h0, coord0], axis=1)
    d0 = x0.shape[1]
    x0p = jnp.zeros((_NPAD, _D0), _F32).at[:_N, :d0].set(x0)
    w10 = jnp.zeros((_D0, _HID), _F32).at[:d0].set(gin0_W1)

    src = edge_index[0]
    dst = edge_index[1]
    owner = dst // _RPO
    perm = jnp.argsort(owner, stable=True)
    owner_s = owner[perm]
    src_s = src[perm]
    dst_s = dst[perm]
    dst_local = (owner_s // 2) * _RPO + (dst_s - owner_s * _RPO)

    cnt_w = jnp.bincount(owner, length=_NW)
    est_w = jnp.concatenate([jnp.zeros((1,), cnt_w.dtype),
                             jnp.cumsum(cnt_w)[:-1]])
    rank = jnp.arange(_E) - est_w[owner_s]
    pos = owner_s * (_CPT * _CH) + rank

    slot_owner = jnp.repeat(jnp.arange(_NW), _CPT)
    pad_dst = jnp.repeat((slot_owner // 2) * _RPO, _CH)
    pad_src = (_N + jnp.arange(_G * _CH) % (_NPAD - _N)).astype(jnp.int32)
    flat_src = pad_src.at[pos].set(src_s)
    flat_dst = pad_dst.astype(jnp.int32).at[pos].set(dst_local)
    e4 = jnp.stack([flat_src.reshape(_NW, _CPT, _CH),
                    flat_dst.reshape(_NW, _CPT, _CH)], axis=2)

    zrows0 = jnp.zeros((_RPO, _D0), _F32)
    zrows = jnp.zeros((_RPO, _HID), _F32)
    batch_pad = jnp.full((_NPAD, 1), _NG, jnp.int32).at[:_N, 0].set(batch)

    def row(v):
        return v.reshape(1, -1)

    gins = [
        (w10, gin0_b1, gin0_gamma, gin0_beta, gin0_W2, gin0_b2),
        (gin1_W1, gin1_b1, gin1_gamma, gin1_beta, gin1_W2, gin1_b2),
        (gin2_W1, gin2_b1, gin2_gamma, gin2_beta, gin2_W2, gin2_b2),
    ]

    x = x0p
    for i in range(3):
        w1, b1, gamma, beta, w2, b2 = gins[i]
        agg = _sc_segsum(x, e4, zrows0 if i == 0 else zrows)
        h2 = _tc_layer(x, agg, w1, row(b1), row(gamma), row(beta), w2,
                       row(b2))
        x = jax.nn.elu(h2, alpha=0.1)

    out = _tc_final(
        x, batch_pad, g0,
        cls_W1[:_HID], cls_W1[_HID:2 * _HID], cls_W1[2 * _HID:],
        row(cls_b1), row(cls_gamma), row(cls_beta), cls_W2, row(cls_b2))
    return out


# speedup vs baseline: 1.4157x; 1.0087x over previous
import functools

import jax
import jax.numpy as jnp
from jax import lax
from jax.experimental import pallas as pl
from jax.experimental.pallas import tpu as pltpu
from jax.experimental.pallas import tpu_sc as plsc

_N = 10000
_NPAD = 10240
_E = 320000
_HID = 128
_D0 = 144
_NG = 16
_NCLS = 8
_NC = 2
_NS = 16
_NW = _NC * _NS
_RPO = _NPAD // _NW
_CH = 128
_CPT = 88
_G = _NW * _CPT

_F32 = jnp.float32


def _dot(a, b):
    return lax.dot_general(a, b, (((1,), (0,)), ((), ())),
                           preferred_element_type=_F32,
                           precision=lax.Precision.DEFAULT)


def _elu(x):
    return jnp.where(x > 0, x, 0.1 * (jnp.exp(jnp.minimum(x, 0.0)) - 1.0))


def _bf(a):
    while a.shape[0] > 1:
        k = a.shape[0] // 2
        a = a[:k] + a[k:]
    return a


def _sc_segsum(x, e4, zrows):
    d = x.shape[1]
    mesh = plsc.VectorSubcoreMesh(core_axis_name="c", subcore_axis_name="s")

    @functools.partial(
        pl.kernel,
        out_type=jax.ShapeDtypeStruct((_NPAD, d), _F32),
        mesh=mesh,
        compiler_params=pltpu.CompilerParams(
            use_tc_tiling_on_sc=(d % 128 == 0)),
        scratch_types=[
            pltpu.VMEM((4, 2, _CH), jnp.int32),
            pltpu.VMEM((4, _CH, d), _F32),
            pltpu.VMEM_SHARED((_NPAD // 2, d), _F32),
            pltpu.SemaphoreType.DMA,
        ],
    )
    def k(x_hbm, e_hbm, z_hbm, out_hbm, idx_v, rows_v, acc_sh, sem):
        c = lax.axis_index("c")
        s = lax.axis_index("s")
        w = s * 2 + c
        row0 = s * _RPO
        pltpu.sync_copy(z_hbm, acc_sh.at[pl.ds(row0, _RPO)])
        plsc.subcore_barrier()

        for p in range(3):
            pltpu.sync_copy(e_hbm.at[w, p], idx_v.at[p])
            pltpu.async_copy(x_hbm.at[idx_v.at[p, 0]], rows_v.at[p], sem)

        def body(j, carry):
            b = lax.rem(j, 4)
            nb = lax.rem(j + 3, 4)

            @pl.when(j + 3 < _CPT)
            def _():
                pltpu.sync_copy(e_hbm.at[w, j + 3], idx_v.at[nb])
                pltpu.async_copy(x_hbm.at[idx_v.at[nb, 0]], rows_v.at[nb],
                                 sem)

            pltpu.make_async_copy(x_hbm.at[idx_v.at[b, 0]], rows_v.at[b],
                                  sem).wait()
            pltpu.sync_copy(rows_v.at[b], acc_sh.at[idx_v.at[b, 1]],
                            add=True)
            return carry

        lax.fori_loop(0, _CPT, body, 0)
        plsc.subcore_barrier()
        pltpu.sync_copy(acc_sh.at[pl.ds(row0, _RPO)],
                        out_hbm.at[pl.ds(w * _RPO, _RPO)])

    return k(x, e4, zrows)


def _stats(h_ref):
    def half(lo):
        def step(i, acc):
            return acc + h_ref[pl.ds(lo + i * 8, 8), :]
        return lax.fori_loop(0, 625, step, jnp.zeros((8, _HID), _F32))

    mu = (_bf(half(0)) + _bf(half(5000))) * (1.0 / _N)

    def halfv(lo):
        def step(i, acc):
            dv = h_ref[pl.ds(lo + i * 8, 8), :] - mu
            return acc + dv * dv
        return lax.fori_loop(0, 625, step, jnp.zeros((8, _HID), _F32))

    var = (_bf(halfv(0)) + _bf(halfv(5000))) * (1.0 / _N)
    return mu, var


def _tc_layer(x, agg, w1, b1, gamma, beta, w2, b2):
    def body(x_ref, a_ref, w1_ref, b1_ref, g_ref, be_ref, w2_ref, b2_ref,
             o_ref, h_s):
        hin = x_ref[...] + a_ref[...]
        h_s[...] = _dot(hin, w1_ref[...]) + b1_ref[...]
        mu, var = _stats(h_s)
        hn = (h_s[...] - mu) / jnp.sqrt(var + 1e-5) * g_ref[...] + be_ref[...]
        hn = jnp.maximum(hn, 0.0)
        h2 = _dot(hn, w2_ref[...]) + b2_ref[...]
        rows = lax.broadcasted_iota(jnp.int32, (_NPAD, 1), 0)
        o_ref[...] = jnp.where(rows < _N, h2, 0.0)

    return pl.pallas_call(
        body,
        out_shape=jax.ShapeDtypeStruct((_NPAD, _HID), _F32),
        scratch_shapes=[pltpu.VMEM((_NPAD, _HID), _F32)],
    )(x, agg, w1, b1, gamma, beta, w2, b2)


def _tc_final(x, batch_pad, g0, cw1a, cw1b, cw1c, cb1, cgamma, cbeta,
              cw2, cb2):
    def body(x_ref, batch_ref, g0_ref, w1a_ref, w1b_ref, w1c_ref, cb1_ref,
             cg_ref, cbe_ref, cw2_ref, cb2_ref, o_ref):
        xf = x_ref[...]
        b = batch_ref[...]
        gids = lax.broadcasted_iota(jnp.int32, (_NPAD, _NG), 1)
        oh = (b == gids).astype(_F32)
        cnt = jnp.sum(oh, axis=0, keepdims=True)
        sums = lax.dot_general(oh, xf, (((0,), (0,)), ((), ())),
                               preferred_element_type=_F32,
                               precision=lax.Precision.HIGHEST)
        mp = sums / jnp.maximum(cnt, 1.0).reshape(_NG, 1)
        mx_list = []
        for g in range(_NG):
            sel = jnp.where(b == g, xf, -jnp.inf)
            mx_list.append(jnp.max(sel, axis=0, keepdims=True))
        mx = jnp.concatenate(mx_list, axis=0)

        z = (_dot(mp, w1a_ref[...]) + _dot(mx, w1b_ref[...])
             + _dot(g0_ref[...], w1c_ref[...]) + cb1_ref[...])
        z = _elu(z)
        zmu = _bf(z) * (1.0 / _NG)
        zd = z - zmu
        zvar = _bf(zd * zd) * (1.0 / _NG)
        zn = zd / jnp.sqrt(zvar + 1e-5) * cg_ref[...] + cbe_ref[...]
        logits = _dot(zn, cw2_ref[...]) + cb2_ref[...]
        lmax = jnp.max(logits, axis=1, keepdims=True)
        e = jnp.exp(logits - lmax)
        o_ref[...] = e / jnp.sum(e, axis=1, keepdims=True)

    return pl.pallas_call(
        body,
        out_shape=jax.ShapeDtypeStruct((_NG, _NCLS), _F32),
    )(x, batch_pad, g0, cw1a, cw1b, cw1c, cb1, cgamma, cbeta, cw2, cb2)


def kernel(h0, coord0, g0, edge_index, batch,
           gin0_W1, gin0_b1, gin0_gamma, gin0_beta, gin0_W2, gin0_b2,
           gin1_W1, gin1_b1, gin1_gamma, gin1_beta, gin1_W2, gin1_b2,
           gin2_W1, gin2_b1, gin2_gamma, gin2_beta, gin2_W2, gin2_b2,
           cls_W1, cls_b1, cls_gamma, cls_beta, cls_W2, cls_b2):
    x0 = jnp.concatenate([h0, coord0], axis=1)
    d0 = x0.shape[1]
    x0p = jnp.zeros((_NPAD, _D0), _F32).at[:_N, :d0].set(x0)
    w10 = jnp.zeros((_D0, _HID), _F32).at[:d0].set(gin0_W1)

    src = edge_index[0]
    dst = edge_index[1]
    owner = dst // _RPO
    perm = jnp.argsort(owner, stable=True)
    owner_s = owner[perm]
    src_s = src[perm]
    dst_s = dst[perm]
    dst_local = (owner_s // 2) * _RPO + (dst_s - owner_s * _RPO)

    cnt_w = jnp.bincount(owner, length=_NW)
    est_w = jnp.concatenate([jnp.zeros((1,), cnt_w.dtype),
                             jnp.cumsum(cnt_w)[:-1]])
    rank = jnp.arange(_E) - est_w[owner_s]
    pos = owner_s * (_CPT * _CH) + rank

    slot_owner = jnp.repeat(jnp.arange(_NW), _CPT)
    pad_dst = jnp.repeat((slot_owner // 2) * _RPO, _CH)
    pad_src = (_N + jnp.arange(_G * _CH) % (_NPAD - _N)).astype(jnp.int32)
    flat_src = pad_src.at[pos].set(src_s)
    flat_dst = pad_dst.astype(jnp.int32).at[pos].set(dst_local)
    e4 = jnp.stack([flat_src.reshape(_NW, _CPT, _CH),
                    flat_dst.reshape(_NW, _CPT, _CH)], axis=2)

    zrows0 = jnp.zeros((_RPO, _D0), _F32)
    zrows = jnp.zeros((_RPO, _HID), _F32)
    batch_pad = jnp.full((_NPAD, 1), _NG, jnp.int32).at[:_N, 0].set(batch)

    def row(v):
        return v.reshape(1, -1)

    gins = [
        (w10, gin0_b1, gin0_gamma, gin0_beta, gin0_W2, gin0_b2),
        (gin1_W1, gin1_b1, gin1_gamma, gin1_beta, gin1_W2, gin1_b2),
        (gin2_W1, gin2_b1, gin2_gamma, gin2_beta, gin2_W2, gin2_b2),
    ]

    x = x0p
    for i in range(3):
        w1, b1, gamma, beta, w2, b2 = gins[i]
        agg = _sc_segsum(x, e4, zrows0 if i == 0 else zrows)
        h2 = _tc_layer(x, agg, w1, row(b1), row(gamma), row(beta), w2,
                       row(b2))
        x = jax.nn.elu(h2, alpha=0.1)

    out = _tc_final(
        x, batch_pad, g0,
        cls_W1[:_HID], cls_W1[_HID:2 * _HID], cls_W1[2 * _HID:],
        row(cls_b1), row(cls_gamma), row(cls_beta), cls_W2, row(cls_b2))
    return out


# speedup vs baseline: 1.6073x; 1.1354x over previous
import functools

import jax
import jax.numpy as jnp
from jax import lax
from jax.experimental import pallas as pl
from jax.experimental.pallas import tpu as pltpu
from jax.experimental.pallas import tpu_sc as plsc

_N = 10000
_NPAD = 10240
_E = 320000
_HID = 128
_D0 = 144
_NG = 16
_NCLS = 8
_NC = 2
_NS = 16
_NW = _NC * _NS
_RPO = _NPAD // _NW
_CH = 128
_CPT = 88
_G = _NW * _CPT

_F32 = jnp.float32


def _dot(a, b):
    return lax.dot_general(a, b, (((1,), (0,)), ((), ())),
                           preferred_element_type=_F32,
                           precision=lax.Precision.DEFAULT)


def _elu(x):
    return jnp.where(x > 0, x, 0.1 * (jnp.exp(jnp.minimum(x, 0.0)) - 1.0))


def _bf(a):
    while a.shape[0] > 1:
        k = a.shape[0] // 2
        a = a[:k] + a[k:]
    return a


def _sc_segsum(x, e4, zrows):
    d = x.shape[1]
    mesh = plsc.VectorSubcoreMesh(core_axis_name="c", subcore_axis_name="s")

    @functools.partial(
        pl.kernel,
        out_type=jax.ShapeDtypeStruct((_NPAD, d), _F32),
        mesh=mesh,
        compiler_params=pltpu.CompilerParams(
            use_tc_tiling_on_sc=(d % 128 == 0)),
        scratch_types=[
            pltpu.VMEM((4, 2, _CH), jnp.int32),
            pltpu.VMEM((4, _CH, d), _F32),
            pltpu.VMEM_SHARED((_NPAD // 2, d), _F32),
            pltpu.SemaphoreType.DMA,
        ],
    )
    def k(x_hbm, e_hbm, z_hbm, out_hbm, idx_v, rows_v, acc_sh, sem):
        c = lax.axis_index("c")
        s = lax.axis_index("s")
        w = s * 2 + c
        row0 = s * _RPO
        pltpu.sync_copy(z_hbm, acc_sh.at[pl.ds(row0, _RPO)])
        plsc.subcore_barrier()

        for p in range(3):
            pltpu.sync_copy(e_hbm.at[w, p], idx_v.at[p])
            pltpu.async_copy(x_hbm.at[idx_v.at[p, 0]], rows_v.at[p], sem)

        def body(j, carry):
            b = lax.rem(j, 4)
            nb = lax.rem(j + 3, 4)

            @pl.when(j + 3 < _CPT)
            def _():
                pltpu.sync_copy(e_hbm.at[w, j + 3], idx_v.at[nb])
                pltpu.async_copy(x_hbm.at[idx_v.at[nb, 0]], rows_v.at[nb],
                                 sem)

            pltpu.make_async_copy(x_hbm.at[idx_v.at[b, 0]], rows_v.at[b],
                                  sem).wait()
            pltpu.sync_copy(rows_v.at[b], acc_sh.at[idx_v.at[b, 1]],
                            add=True)
            return carry

        lax.fori_loop(0, _CPT, body, 0)
        plsc.subcore_barrier()
        pltpu.sync_copy(acc_sh.at[pl.ds(row0, _RPO)],
                        out_hbm.at[pl.ds(w * _RPO, _RPO)])

    return k(x, e4, zrows)


def _stats(h_ref):
    def half(lo):
        def step(i, acc):
            return acc + h_ref[pl.ds(lo + i * 8, 8), :]
        return lax.fori_loop(0, 625, step, jnp.zeros((8, _HID), _F32))

    mu = (_bf(half(0)) + _bf(half(5000))) * (1.0 / _N)

    def halfv(lo):
        def step(i, acc):
            dv = h_ref[pl.ds(lo + i * 8, 8), :] - mu
            return acc + dv * dv
        return lax.fori_loop(0, 625, step, jnp.zeros((8, _HID), _F32))

    var = (_bf(halfv(0)) + _bf(halfv(5000))) * (1.0 / _N)
    return mu, var


def _tc_layer(x, agg, w1, b1, gamma, beta, w2, b2):
    def body(x_ref, a_ref, w1_ref, b1_ref, g_ref, be_ref, w2_ref, b2_ref,
             o_ref, h_s):
        hin = x_ref[...] + a_ref[...]
        h_s[...] = _dot(hin, w1_ref[...]) + b1_ref[...]
        mu, var = _stats(h_s)
        hn = (h_s[...] - mu) / jnp.sqrt(var + 1e-5) * g_ref[...] + be_ref[...]
        hn = jnp.maximum(hn, 0.0)
        h2 = _dot(hn, w2_ref[...]) + b2_ref[...]
        rows = lax.broadcasted_iota(jnp.int32, (_NPAD, 1), 0)
        o_ref[...] = jnp.where(rows < _N, h2, 0.0)

    return pl.pallas_call(
        body,
        out_shape=jax.ShapeDtypeStruct((_NPAD, _HID), _F32),
        scratch_shapes=[pltpu.VMEM((_NPAD, _HID), _F32)],
    )(x, agg, w1, b1, gamma, beta, w2, b2)


def _tc_final(x, batch_pad, g0, cw1a, cw1b, cw1c, cb1, cgamma, cbeta,
              cw2, cb2):
    def body(x_ref, batch_ref, g0_ref, w1a_ref, w1b_ref, w1c_ref, cb1_ref,
             cg_ref, cbe_ref, cw2_ref, cb2_ref, o_ref):
        xf = x_ref[...]
        b = batch_ref[...]
        gids = lax.broadcasted_iota(jnp.int32, (_NPAD, _NG), 1)
        oh = (b == gids).astype(_F32)
        cnt = jnp.sum(oh, axis=0, keepdims=True)
        sums = lax.dot_general(oh, xf, (((0,), (0,)), ((), ())),
                               preferred_element_type=_F32,
                               precision=lax.Precision.HIGHEST)
        mp = sums / jnp.maximum(cnt, 1.0).reshape(_NG, 1)
        mx_list = []
        for g in range(_NG):
            sel = jnp.where(b == g, xf, -jnp.inf)
            mx_list.append(jnp.max(sel, axis=0, keepdims=True))
        mx = jnp.concatenate(mx_list, axis=0)

        z = (_dot(mp, w1a_ref[...]) + _dot(mx, w1b_ref[...])
             + _dot(g0_ref[...], w1c_ref[...]) + cb1_ref[...])
        z = _elu(z)
        zmu = _bf(z) * (1.0 / _NG)
        zd = z - zmu
        zvar = _bf(zd * zd) * (1.0 / _NG)
        zn = zd / jnp.sqrt(zvar + 1e-5) * cg_ref[...] + cbe_ref[...]
        logits = _dot(zn, cw2_ref[...]) + cb2_ref[...]
        lmax = jnp.max(logits, axis=1, keepdims=True)
        e = jnp.exp(logits - lmax)
        o_ref[...] = e / jnp.sum(e, axis=1, keepdims=True)

    return pl.pallas_call(
        body,
        out_shape=jax.ShapeDtypeStruct((_NG, _NCLS), _F32),
    )(x, batch_pad, g0, cw1a, cw1b, cw1c, cb1, cgamma, cbeta, cw2, cb2)


def kernel(h0, coord0, g0, edge_index, batch,
           gin0_W1, gin0_b1, gin0_gamma, gin0_beta, gin0_W2, gin0_b2,
           gin1_W1, gin1_b1, gin1_gamma, gin1_beta, gin1_W2, gin1_b2,
           gin2_W1, gin2_b1, gin2_gamma, gin2_beta, gin2_W2, gin2_b2,
           cls_W1, cls_b1, cls_gamma, cls_beta, cls_W2, cls_b2):
    x0 = jnp.concatenate([h0, coord0], axis=1)
    d0 = x0.shape[1]
    x0p = jnp.zeros((_NPAD, _D0), _F32).at[:_N, :d0].set(x0)
    w10 = jnp.zeros((_D0, _HID), _F32).at[:d0].set(gin0_W1)

    src = edge_index[0]
    dst = edge_index[1]
    owner = dst // _RPO
    dst_local = (owner // 2) * _RPO + (dst - owner * _RPO)

    oh = (owner[:, None] == jnp.arange(_NW)[None, :]).astype(jnp.int32)
    rank = jnp.take_along_axis(jnp.cumsum(oh, axis=0), owner[:, None],
                               axis=1)[:, 0] - 1
    pos = owner * (_CPT * _CH) + rank
    src_s = src

    slot_owner = jnp.repeat(jnp.arange(_NW), _CPT)
    pad_dst = jnp.repeat((slot_owner // 2) * _RPO, _CH)
    pad_src = (_N + jnp.arange(_G * _CH) % (_NPAD - _N)).astype(jnp.int32)
    flat_src = pad_src.at[pos].set(src_s)
    flat_dst = pad_dst.astype(jnp.int32).at[pos].set(dst_local)
    e4 = jnp.stack([flat_src.reshape(_NW, _CPT, _CH),
                    flat_dst.reshape(_NW, _CPT, _CH)], axis=2)

    zrows0 = jnp.zeros((_RPO, _D0), _F32)
    zrows = jnp.zeros((_RPO, _HID), _F32)
    batch_pad = jnp.full((_NPAD, 1), _NG, jnp.int32).at[:_N, 0].set(batch)

    def row(v):
        return v.reshape(1, -1)

    gins = [
        (w10, gin0_b1, gin0_gamma, gin0_beta, gin0_W2, gin0_b2),
        (gin1_W1, gin1_b1, gin1_gamma, gin1_beta, gin1_W2, gin1_b2),
        (gin2_W1, gin2_b1, gin2_gamma, gin2_beta, gin2_W2, gin2_b2),
    ]

    x = x0p
    for i in range(3):
        w1, b1, gamma, beta, w2, b2 = gins[i]
        agg = _sc_segsum(x, e4, zrows0 if i == 0 else zrows)
        h2 = _tc_layer(x, agg, w1, row(b1), row(gamma), row(beta), w2,
                       row(b2))
        x = jax.nn.elu(h2, alpha=0.1)

    out = _tc_final(
        x, batch_pad, g0,
        cls_W1[:_HID], cls_W1[_HID:2 * _HID], cls_W1[2 * _HID:],
        row(cls_b1), row(cls_gamma), row(cls_beta), cls_W2, row(cls_b2))
    return out
